# baseline (speedup 1.0000x reference)
import functools

import jax
import jax.numpy as jnp
from jax import lax
from jax.experimental import pallas as pl
from jax.experimental.pallas import tpu as pltpu
from jax.experimental.pallas import tpu_sc as plsc

B, L, K_EDGE = 16, 2048, 30
MIN_B, MAX_B = 0.1, 20.0
NC, NS = 2, 16
HALF = L // 2


def _sc_shift_body(ca_hbm, nca_hbm, cca_hbm, scal_hbm,
                   ca_out, nca_out, cca_out,
                   rowa, rowb, rowc, scal_v, outa, outb, outc, sem):
    wid = lax.axis_index("s") * NC + lax.axis_index("c")
    b = wid // 2
    h = wid % 2
    cps = []
    for c in range(3):
        cps.append(pltpu.async_copy(ca_hbm.at[c, b], rowa.at[pl.ds(c * L, L)], sem))
        cps.append(pltpu.async_copy(nca_hbm.at[c, b], rowb.at[pl.ds(c * L, L)], sem))
        cps.append(pltpu.async_copy(cca_hbm.at[c, b], rowc.at[pl.ds(c * L, L)], sem))
    cps.append(pltpu.async_copy(scal_hbm, scal_v, sem))
    for cp in cps:
        cp.wait()
    len_b = scal_v[pl.ds(b, 16)][0]
    rs_b = scal_v[pl.ds(b + 16, 16)][0]
    roll = scal_v[pl.ds(32, 16)][0]
    base = h * HALF
    iota = lax.broadcasted_iota(jnp.int32, (16,), 0)

    def chunk(ci, carry):
        i = base + ci * 16 + iota
        jm = lax.rem(lax.rem(i - roll, L) + L, L)
        k = jnp.minimum(jnp.maximum(jm - rs_b, 0), len_b - 1)
        off = ci * 16
        for c in range(3):
            kc = k + c * L
            outa[pl.ds(c * HALF + off, 16)] = plsc.load_gather(rowa, [kc])
            outb[pl.ds(c * HALF + off, 16)] = plsc.load_gather(rowb, [kc])
            outc[pl.ds(c * HALF + off, 16)] = plsc.load_gather(rowc, [kc])
        return carry

    lax.fori_loop(0, HALF // 16, chunk, 0)
    ops = []
    for c in range(3):
        ops.append(pltpu.async_copy(outa.at[pl.ds(c * HALF, HALF)],
                                    ca_out.at[c, b, pl.ds(base, HALF)], sem))
        ops.append(pltpu.async_copy(outb.at[pl.ds(c * HALF, HALF)],
                                    nca_out.at[c, b, pl.ds(base, HALF)], sem))
        ops.append(pltpu.async_copy(outc.at[pl.ds(c * HALF, HALF)],
                                    cca_out.at[c, b, pl.ds(base, HALF)], sem))
    for op in ops:
        op.wait()


@functools.cache
def _sc_shift():
    return pl.kernel(
        _sc_shift_body,
        out_type=(jax.ShapeDtypeStruct((3, B, L), jnp.float32),) * 3,
        mesh=plsc.VectorSubcoreMesh(core_axis_name="c", subcore_axis_name="s",
                                    num_cores=NC, num_subcores=NS),
        compiler_params=pltpu.CompilerParams(needs_layout_passes=False),
        scratch_types=[
            pltpu.VMEM((3 * L,), jnp.float32),
            pltpu.VMEM((3 * L,), jnp.float32),
            pltpu.VMEM((3 * L,), jnp.float32),
            pltpu.VMEM((3 * B,), jnp.int32),
            pltpu.VMEM((3 * HALF,), jnp.float32),
            pltpu.VMEM((3 * HALF,), jnp.float32),
            pltpu.VMEM((3 * HALF,), jnp.float32),
            pltpu.SemaphoreType.DMA,
        ],
    )


def _alpha_sigma(t):
    int_beta = t * MIN_B + 0.5 * t * t * (MAX_B - MIN_B)
    alpha = jnp.exp(-0.5 * int_beta)
    sigma = jnp.sqrt(1.0 - jnp.exp(-int_beta))
    return alpha, sigma


def _edges_body(t_ref, noise_ref, out_ref):
    alpha, sigma = _alpha_sigma(t_ref[pl.program_id(0), 0])
    ch = lax.broadcasted_iota(jnp.int32, out_ref.shape, 2)
    mask = (ch == 1).astype(jnp.float32)
    out_ref[...] = sigma * noise_ref[...] + alpha * mask


def _backbone_body(t_ref, ca_ref, nc_ref, cc_ref, rot_ref, nca_ref,
                   can_ref, ncn_ref, ccn_ref, ss_ref):
    alpha, sigma = _alpha_sigma(t_ref[...])
    ss_ref[...] = 1.0 / sigma
    vx, vy, vz = rot_ref[0], rot_ref[1], rot_ref[2]
    theta = jnp.sqrt(vx * vx + vy * vy + vz * vz)
    safe = jnp.where(theta < 1e-8, 1.0, theta)
    inv = 1.0 / safe
    kx, ky, kz = vx * inv, vy * inv, vz * inv
    sn = jnp.sin(theta)
    c1 = 1.0 - jnp.cos(theta)

    def rodrigues(ref, oref):
        x, y, z = ref[0], ref[1], ref[2]
        cx = ky * z - kz * y
        cy = kz * x - kx * z
        cz = kx * y - ky * x
        dx = ky * cz - kz * cy
        dy = kz * cx - kx * cz
        dz = kx * cy - ky * cx
        oref[0] = x + sn * cx + c1 * dx
        oref[1] = y + sn * cy + c1 * dy
        oref[2] = z + sn * cz + c1 * dz

    rodrigues(nc_ref, ncn_ref)
    rodrigues(cc_ref, ccn_ref)
    for c in range(3):
        can_ref[c] = alpha * ca_ref[c] + sigma * nca_ref[c]


def kernel(ca, n_ca, c_ca, lengths, randstart, randroll, t_vec, rot_vec,
           noise_ca, noise_edges):
    scal = jnp.concatenate([lengths.astype(jnp.int32),
                            randstart.astype(jnp.int32),
                            jnp.full((B,), randroll, dtype=jnp.int32)])
    t_col = t_vec.reshape(B, 1)

    ca_t = ca.transpose(2, 0, 1)
    nca_t = n_ca.transpose(2, 0, 1)
    cca_t = c_ca.transpose(2, 0, 1)

    ca_s, nc_s, cc_s = _sc_shift()(ca_t, nca_t, cca_t, scal)

    noise_e = noise_edges.transpose(0, 2, 3, 4, 1).reshape(B, K_EDGE * 3, 2, L)
    edges_n = pl.pallas_call(
        _edges_body,
        grid=(B, 2),
        in_specs=[pl.BlockSpec((B, 1), lambda g, l: (0, 0),
                               memory_space=pltpu.SMEM),
                  pl.BlockSpec((1, K_EDGE * 3, 2, L // 2),
                               lambda g, l: (g, 0, 0, l))],
        out_specs=pl.BlockSpec((1, K_EDGE * 3, 2, L // 2),
                               lambda g, l: (g, 0, 0, l)),
        out_shape=jax.ShapeDtypeStruct((B, K_EDGE * 3, 2, L), jnp.float32),
    )(t_col, noise_e)

    rot3 = rot_vec.reshape(B, L, 3).transpose(2, 0, 1)
    noise3 = noise_ca.transpose(2, 0, 1)
    plane = jax.ShapeDtypeStruct((3, B, L), jnp.float32)
    can, ncn, ccn, ss = pl.pallas_call(
        _backbone_body,
        out_shape=(plane, plane, plane,
                   jax.ShapeDtypeStruct((B, 1), jnp.float32)),
    )(t_col, ca_s, nc_s, cc_s, rot3, noise3)

    ca_noised = can.transpose(1, 2, 0)
    nc_noised = ncn.transpose(1, 2, 0)
    cc_noised = ccn.transpose(1, 2, 0)
    score_scales = ss.reshape(B)
    edges_noised = edges_n.reshape(B, K_EDGE, 3, 2, L).transpose(0, 4, 1, 2, 3)
    return (ca_noised, nc_noised, cc_noised, t_vec, score_scales, edges_noised)


# speedup vs baseline: 1.2545x; 1.2545x over previous
import functools

import jax
import jax.numpy as jnp
from jax import lax
from jax.experimental import pallas as pl
from jax.experimental.pallas import tpu as pltpu
from jax.experimental.pallas import tpu_sc as plsc

B, L, K_EDGE = 16, 2048, 30
MIN_B, MAX_B = 0.1, 20.0
NC, NS = 2, 16
HALF = L // 2


def _sc_shift_body(ca_hbm, nca_hbm, cca_hbm, scal_hbm,
                   ca_out, nca_out, cca_out,
                   rowa, rowb, rowc, scal_v, outa, outb, outc, sem):
    wid = lax.axis_index("s") * NC + lax.axis_index("c")
    b = wid // 2
    h = wid % 2
    cps = []
    for c in range(3):
        cps.append(pltpu.async_copy(ca_hbm.at[c, b], rowa.at[pl.ds(c * L, L)], sem))
        cps.append(pltpu.async_copy(nca_hbm.at[c, b], rowb.at[pl.ds(c * L, L)], sem))
        cps.append(pltpu.async_copy(cca_hbm.at[c, b], rowc.at[pl.ds(c * L, L)], sem))
    cps.append(pltpu.async_copy(scal_hbm, scal_v, sem))
    for cp in cps:
        cp.wait()
    len_b = scal_v[pl.ds(b, 16)][0]
    rs_b = scal_v[pl.ds(b + 16, 16)][0]
    roll = scal_v[pl.ds(32, 16)][0]
    base = h * HALF
    iota = lax.broadcasted_iota(jnp.int32, (16,), 0)

    def chunk(ci, carry):
        i = base + ci * 16 + iota
        jm = lax.rem(lax.rem(i - roll, L) + L, L)
        k = jnp.minimum(jnp.maximum(jm - rs_b, 0), len_b - 1)
        off = ci * 16
        for c in range(3):
            kc = k + c * L
            outa[pl.ds(c * HALF + off, 16)] = plsc.load_gather(rowa, [kc])
            outb[pl.ds(c * HALF + off, 16)] = plsc.load_gather(rowb, [kc])
            outc[pl.ds(c * HALF + off, 16)] = plsc.load_gather(rowc, [kc])
        return carry

    lax.fori_loop(0, HALF // 16, chunk, 0)
    ops = []
    for c in range(3):
        ops.append(pltpu.async_copy(outa.at[pl.ds(c * HALF, HALF)],
                                    ca_out.at[c, b, pl.ds(base, HALF)], sem))
        ops.append(pltpu.async_copy(outb.at[pl.ds(c * HALF, HALF)],
                                    nca_out.at[c, b, pl.ds(base, HALF)], sem))
        ops.append(pltpu.async_copy(outc.at[pl.ds(c * HALF, HALF)],
                                    cca_out.at[c, b, pl.ds(base, HALF)], sem))
    for op in ops:
        op.wait()


@functools.cache
def _sc_shift():
    return pl.kernel(
        _sc_shift_body,
        out_type=(jax.ShapeDtypeStruct((3, B, L), jnp.float32),) * 3,
        mesh=plsc.VectorSubcoreMesh(core_axis_name="c", subcore_axis_name="s",
                                    num_cores=NC, num_subcores=NS),
        compiler_params=pltpu.CompilerParams(needs_layout_passes=False),
        scratch_types=[
            pltpu.VMEM((3 * L,), jnp.float32),
            pltpu.VMEM((3 * L,), jnp.float32),
            pltpu.VMEM((3 * L,), jnp.float32),
            pltpu.VMEM((3 * B,), jnp.int32),
            pltpu.VMEM((3 * HALF,), jnp.float32),
            pltpu.VMEM((3 * HALF,), jnp.float32),
            pltpu.VMEM((3 * HALF,), jnp.float32),
            pltpu.SemaphoreType.DMA,
        ],
    )


def _alpha_sigma(t):
    int_beta = t * MIN_B + 0.5 * t * t * (MAX_B - MIN_B)
    alpha = jnp.exp(-0.5 * int_beta)
    sigma = jnp.sqrt(1.0 - jnp.exp(-int_beta))
    return alpha, sigma


EDGE_NB = 2


def _edges_body(t_ref, noise_ref, out_ref):
    ch = lax.broadcasted_iota(jnp.int32, out_ref.shape[1:], 1)
    mask = (ch == 1).astype(jnp.float32)
    g = pl.program_id(0)
    for i in range(EDGE_NB):
        alpha, sigma = _alpha_sigma(t_ref[g * EDGE_NB + i, 0])
        out_ref[i] = sigma * noise_ref[i] + alpha * mask


def _backbone_body(t_ref, ca_ref, nc_ref, cc_ref, rot_ref, nca_ref,
                   can_ref, ncn_ref, ccn_ref, ss_ref):
    alpha, sigma = _alpha_sigma(t_ref[...])
    ss_ref[...] = 1.0 / sigma
    vx, vy, vz = rot_ref[0], rot_ref[1], rot_ref[2]
    theta = jnp.sqrt(vx * vx + vy * vy + vz * vz)
    safe = jnp.where(theta < 1e-8, 1.0, theta)
    inv = 1.0 / safe
    kx, ky, kz = vx * inv, vy * inv, vz * inv
    sn = jnp.sin(theta)
    c1 = 1.0 - jnp.cos(theta)

    def rodrigues(ref, oref):
        x, y, z = ref[0], ref[1], ref[2]
        cx = ky * z - kz * y
        cy = kz * x - kx * z
        cz = kx * y - ky * x
        dx = ky * cz - kz * cy
        dy = kz * cx - kx * cz
        dz = kx * cy - ky * cx
        oref[0] = x + sn * cx + c1 * dx
        oref[1] = y + sn * cy + c1 * dy
        oref[2] = z + sn * cz + c1 * dz

    rodrigues(nc_ref, ncn_ref)
    rodrigues(cc_ref, ccn_ref)
    for c in range(3):
        can_ref[c] = alpha * ca_ref[c] + sigma * nca_ref[c]


def kernel(ca, n_ca, c_ca, lengths, randstart, randroll, t_vec, rot_vec,
           noise_ca, noise_edges):
    scal = jnp.concatenate([lengths.astype(jnp.int32),
                            randstart.astype(jnp.int32),
                            jnp.full((B,), randroll, dtype=jnp.int32)])
    t_col = t_vec.reshape(B, 1)

    ca_t = ca.transpose(2, 0, 1)
    nca_t = n_ca.transpose(2, 0, 1)
    cca_t = c_ca.transpose(2, 0, 1)

    ca_s, nc_s, cc_s = _sc_shift()(ca_t, nca_t, cca_t, scal)

    noise_e = noise_edges.transpose(0, 2, 3, 4, 1).reshape(B, K_EDGE * 3, 2, L)
    edges_n = pl.pallas_call(
        _edges_body,
        grid=(B // EDGE_NB,),
        in_specs=[pl.BlockSpec((B, 1), lambda g: (0, 0),
                               memory_space=pltpu.SMEM),
                  pl.BlockSpec((EDGE_NB, K_EDGE * 3, 2, L),
                               lambda g: (g, 0, 0, 0))],
        out_specs=pl.BlockSpec((EDGE_NB, K_EDGE * 3, 2, L),
                               lambda g: (g, 0, 0, 0)),
        out_shape=jax.ShapeDtypeStruct((B, K_EDGE * 3, 2, L), jnp.float32),
    )(t_col, noise_e)

    rot3 = rot_vec.reshape(B, L, 3).transpose(2, 0, 1)
    noise3 = noise_ca.transpose(2, 0, 1)
    plane = jax.ShapeDtypeStruct((3, B, L), jnp.float32)
    can, ncn, ccn, ss = pl.pallas_call(
        _backbone_body,
        out_shape=(plane, plane, plane,
                   jax.ShapeDtypeStruct((B, 1), jnp.float32)),
    )(t_col, ca_s, nc_s, cc_s, rot3, noise3)

    ca_noised = can.transpose(1, 2, 0)
    nc_noised = ncn.transpose(1, 2, 0)
    cc_noised = ccn.transpose(1, 2, 0)
    score_scales = ss.reshape(B)
    edges_noised = edges_n.reshape(B, K_EDGE, 3, 2, L).transpose(0, 4, 1, 2, 3)
    return (ca_noised, nc_noised, cc_noised, t_vec, score_scales, edges_noised)
